# baseline (speedup 1.0000x reference)
import jax
import jax.numpy as jnp
from jax import lax
from jax.experimental import pallas as pl
from jax.experimental.pallas import tpu as pltpu
from jax.experimental.pallas import tpu_sc as plsc

_B = 16384
_L = 20
_E = 64
_N = _B * _L
_CHUNK = 128
_NW = 32
_NCHUNK = _N // _CHUNK
_CPW = _NCHUNK // _NW
_NET = 9
_TSLAB = _NET * 8 * 128
_KTILES = _B // 128
_LSTRIDE = _NET * _KTILES * 1024


def _hash_embed_body(ids_hbm, hcol0_hbm, hcol1_hbm, w_hbm, pcol0_hbm,
                     pcol1_hbm, out_hbm,
                     ids_v, h0_v, h1_v, h0h_v, h1h_v, p0_v, p1_v,
                     pv0_v, pv1_v, w0_v, w1_v, outb_v,
                     sem_a, sem_b, sem_out):
  wid = lax.axis_index("s") * 2 + lax.axis_index("c")
  row0 = wid * _CPW

  pltpu.sync_copy(ids_hbm.at[pl.ds(row0, _CPW)], ids_v)

  iota16 = lax.iota(jnp.int32, 16)

  def start_a(c):
    s = lax.rem(c, 3)
    idx = ids_v.at[c]
    pltpu.async_copy(hcol0_hbm.at[idx], h0_v.at[s], sem_a)
    pltpu.async_copy(hcol1_hbm.at[idx], h1_v.at[s], sem_a)
    pltpu.async_copy(pcol0_hbm.at[idx], p0_v.at[s], sem_a)
    pltpu.async_copy(pcol1_hbm.at[idx], p1_v.at[s], sem_a)

  def wait_a():
    idx = ids_v.at[0]
    pltpu.make_async_copy(hcol0_hbm.at[idx], h0_v.at[0], sem_a).wait()
    pltpu.make_async_copy(hcol1_hbm.at[idx], h1_v.at[0], sem_a).wait()
    pltpu.make_async_copy(pcol0_hbm.at[idx], p0_v.at[0], sem_a).wait()
    pltpu.make_async_copy(pcol1_hbm.at[idx], p1_v.at[0], sem_a).wait()

  def start_b(c):
    s = lax.rem(c, 3)
    b = c & 1
    for g in range(_CHUNK // 16):
      sl = pl.ds(g * 16, 16)
      h0h_v[s, sl] = lax.shift_right_logical(h0_v[s, sl], 1)
      h1h_v[s, sl] = lax.shift_right_logical(h1_v[s, sl], 1)
    pltpu.async_copy(w_hbm.at[h0h_v.at[s]], w0_v.at[b], sem_b)
    pltpu.async_copy(w_hbm.at[h1h_v.at[s]], w1_v.at[b], sem_b)
    pltpu.async_copy(pcol0_hbm.at[h0_v.at[s]], pv0_v.at[b], sem_b)
    pltpu.async_copy(pcol1_hbm.at[h1_v.at[s]], pv1_v.at[b], sem_b)

  def wait_b():
    pltpu.make_async_copy(w_hbm.at[h0h_v.at[0]], w0_v.at[0], sem_b).wait()
    pltpu.make_async_copy(w_hbm.at[h1h_v.at[0]], w1_v.at[0], sem_b).wait()
    pltpu.make_async_copy(pcol0_hbm.at[h0_v.at[0]], pv0_v.at[0], sem_b).wait()
    pltpu.make_async_copy(pcol1_hbm.at[h1_v.at[0]], pv1_v.at[0], sem_b).wait()

  def start_d(c):
    boff = (c & 1) * _TSLAB
    r = row0 + c
    l = lax.div(r, _KTILES)
    k = lax.rem(r, _KTILES)
    base = l * _LSTRIDE + k * 1024
    for t in range(_NET):
      pltpu.async_copy(outb_v.at[pl.ds(boff + t * 1024, 1024)],
                       out_hbm.at[pl.ds(base + t * 131072, 1024)], sem_out)

  def wait_d():
    for t in range(_NET):
      pltpu.make_async_copy(outb_v.at[pl.ds(0, 1024)],
                            out_hbm.at[pl.ds(0, 1024)], sem_out).wait()

  def compute(c):
    s = lax.rem(c, 3)
    b = c & 1
    boff = b * _TSLAB
    w0b = w0_v.at[b]
    w1b = w1_v.at[b]

    def group_body(g, carry2):
      base = g * 16
      toks = iota16 + base
      p0vec = p0_v[s, pl.ds(base, 16)]
      p1vec = p1_v[s, pl.ds(base, 16)]
      col0 = (h0_v[s, pl.ds(base, 16)] & 1) * _E
      col1 = (h1_v[s, pl.ds(base, 16)] & 1) * _E
      sbase = boff + base

      @plsc.parallel_loop(0, _E, step=1, unroll=8)
      def _(e):
        g0 = plsc.load_gather(w0b, [toks, col0 + e])
        g1 = plsc.load_gather(w1b, [toks, col1 + e])
        outb_v[pl.ds(sbase + e * 128, 16)] = g0 * p0vec + g1 * p1vec

      pb = boff + 8 * 1024 + base
      outb_v[pl.ds(pb, 16)] = pv0_v[b, pl.ds(base, 16)]
      outb_v[pl.ds(pb + 128, 16)] = pv1_v[b, pl.ds(base, 16)]
      return carry2

    lax.fori_loop(0, _CHUNK // 16, group_body, 0)

  start_a(jnp.int32(0))
  start_a(jnp.int32(1))
  wait_a()
  start_b(jnp.int32(0))

  def chunk_body(c, carry):
    @pl.when(c < _CPW - 2)
    def _():
      start_a(c + 2)

    @pl.when(c < _CPW - 1)
    def _():
      wait_a()
      start_b(c + 1)

    wait_b()

    @pl.when(c >= 2)
    def _():
      wait_d()

    compute(c)
    start_d(c)
    return carry

  lax.fori_loop(0, _CPW, chunk_body, 0)
  wait_d()
  wait_d()


def kernel(words_as_ids, hash_table, W, P):
  ids = words_as_ids.astype(jnp.int32).T.reshape(_NCHUNK, _CHUNK)
  hcol0 = hash_table[:, 0].astype(jnp.int32)
  hcol1 = hash_table[:, 1].astype(jnp.int32)
  pcol0 = P[:, 0]
  pcol1 = P[:, 1]
  w2 = W.reshape(W.shape[0] // 2, 2 * _E)

  mesh = plsc.VectorSubcoreMesh(core_axis_name="c", subcore_axis_name="s")
  run = pl.kernel(
      _hash_embed_body,
      out_type=jax.ShapeDtypeStruct((_L * _LSTRIDE,), jnp.float32),
      mesh=mesh,
      compiler_params=pltpu.CompilerParams(needs_layout_passes=False),
      scratch_types=[
          pltpu.VMEM((_CPW, _CHUNK), jnp.int32),
          pltpu.VMEM((3, _CHUNK), jnp.int32),
          pltpu.VMEM((3, _CHUNK), jnp.int32),
          pltpu.VMEM((3, _CHUNK), jnp.int32),
          pltpu.VMEM((3, _CHUNK), jnp.int32),
          pltpu.VMEM((3, _CHUNK), jnp.float32),
          pltpu.VMEM((3, _CHUNK), jnp.float32),
          pltpu.VMEM((2, _CHUNK), jnp.float32),
          pltpu.VMEM((2, _CHUNK), jnp.float32),
          pltpu.VMEM((2, _CHUNK, 2 * _E), jnp.float32),
          pltpu.VMEM((2, _CHUNK, 2 * _E), jnp.float32),
          pltpu.VMEM((2 * _TSLAB,), jnp.float32),
          pltpu.SemaphoreType.DMA,
          pltpu.SemaphoreType.DMA,
          pltpu.SemaphoreType.DMA,
      ],
  )
  raw = run(ids, hcol0, hcol1, w2, pcol0, pcol1)
  out = raw.reshape(_L, _NET, _KTILES, 8, _CHUNK)
  out = out.transpose(2, 4, 0, 1, 3).reshape(_B, _L, _NET * 8)
  return out[:, :, :_E + 2]


# speedup vs baseline: 1.6373x; 1.6373x over previous
import jax
import jax.numpy as jnp
from jax import lax
from jax.experimental import pallas as pl
from jax.experimental.pallas import tpu as pltpu
from jax.experimental.pallas import tpu_sc as plsc

_B = 16384
_L = 20
_E = 64
_N = _B * _L
_CHUNK = 128
_NW = 32
_NCHUNK = _N // _CHUNK
_CPW = _NCHUNK // _NW
_NET = 9
_WPAD = 2 * _E + 4
_OSLAB = 8 * 1024 + 2 * 128
_KTILES = _B // 128
_LSTRIDE = _NET * _KTILES * 1024


def _hash_embed_body(ids_hbm, hcol0_hbm, hcol1_hbm, w_hbm, pcol0_hbm,
                     pcol1_hbm, out_hbm,
                     ids3_v, h0_v, h1_v, h0h_v, h1h_v, p0_v, p1_v,
                     pv0_v, pv1_v, w0_v, w1_v, tm_v, outb_v,
                     sem_i, sem_a, sem_b, sem_out):
  wid = lax.axis_index("s") * 2 + lax.axis_index("c")
  row0 = wid * _CPW

  iota16 = lax.iota(jnp.int32, 16)

  def start_i(c):
    pltpu.async_copy(ids_hbm.at[row0 + c], ids3_v.at[lax.rem(c, 3)], sem_i)

  def wait_i():
    pltpu.make_async_copy(ids_hbm.at[0], ids3_v.at[0], sem_i).wait()

  def start_a(c):
    s = lax.rem(c, 3)
    idx = ids3_v.at[s]
    pltpu.async_copy(hcol0_hbm.at[idx], h0_v.at[s], sem_a)
    pltpu.async_copy(hcol1_hbm.at[idx], h1_v.at[s], sem_a)
    pltpu.async_copy(pcol0_hbm.at[idx], p0_v.at[s], sem_a)
    pltpu.async_copy(pcol1_hbm.at[idx], p1_v.at[s], sem_a)

  def wait_a():
    idx = ids3_v.at[0]
    pltpu.make_async_copy(hcol0_hbm.at[idx], h0_v.at[0], sem_a).wait()
    pltpu.make_async_copy(hcol1_hbm.at[idx], h1_v.at[0], sem_a).wait()
    pltpu.make_async_copy(pcol0_hbm.at[idx], p0_v.at[0], sem_a).wait()
    pltpu.make_async_copy(pcol1_hbm.at[idx], p1_v.at[0], sem_a).wait()

  def start_b(c):
    s = lax.rem(c, 3)
    b = c & 1
    for g in range(_CHUNK // 16):
      sl = pl.ds(g * 16, 16)
      h0h_v[s, sl] = lax.shift_right_logical(h0_v[s, sl], 1)
      h1h_v[s, sl] = lax.shift_right_logical(h1_v[s, sl], 1)
    pltpu.async_copy(w_hbm.at[h0h_v.at[s]], w0_v.at[b], sem_b)
    pltpu.async_copy(w_hbm.at[h1h_v.at[s]], w1_v.at[b], sem_b)
    pltpu.async_copy(pcol0_hbm.at[h0_v.at[s]], pv0_v.at[b], sem_b)
    pltpu.async_copy(pcol1_hbm.at[h1_v.at[s]], pv1_v.at[b], sem_b)

  def wait_b():
    pltpu.make_async_copy(w_hbm.at[h0h_v.at[0]], w0_v.at[0], sem_b).wait()
    pltpu.make_async_copy(w_hbm.at[h1h_v.at[0]], w1_v.at[0], sem_b).wait()
    pltpu.make_async_copy(pcol0_hbm.at[h0_v.at[0]], pv0_v.at[0], sem_b).wait()
    pltpu.make_async_copy(pcol1_hbm.at[h1_v.at[0]], pv1_v.at[0], sem_b).wait()

  def start_d(c):
    boff = (c & 1) * _OSLAB
    r = row0 + c
    l = lax.div(r, _KTILES)
    k = lax.rem(r, _KTILES)
    base = l * _LSTRIDE + k * 1024
    for t in range(8):
      pltpu.async_copy(outb_v.at[pl.ds(boff + t * 1024, 1024)],
                       out_hbm.at[pl.ds(base + t * 131072, 1024)], sem_out)
    pltpu.async_copy(outb_v.at[pl.ds(boff + 8 * 1024, 256)],
                     out_hbm.at[pl.ds(base + 8 * 131072, 256)], sem_out)

  def wait_d():
    for t in range(8):
      pltpu.make_async_copy(outb_v.at[pl.ds(0, 1024)],
                            out_hbm.at[pl.ds(0, 1024)], sem_out).wait()
    pltpu.make_async_copy(outb_v.at[pl.ds(0, 256)],
                          out_hbm.at[pl.ds(0, 256)], sem_out).wait()

  def compute(c):
    s = lax.rem(c, 3)
    b = c & 1
    boff = b * _OSLAB

    def tok_group(g, carry2):
      base = g * 16
      p0vec = p0_v[s, pl.ds(base, 16)]
      p1vec = p1_v[s, pl.ds(base, 16)]
      off0vec = (h0_v[s, pl.ds(base, 16)] & 1) * _E
      off1vec = (h1_v[s, pl.ds(base, 16)] & 1) * _E
      for r16 in range(16):
        tok = base + r16
        p0s = p0vec[r16]
        p1s = p1vec[r16]
        off0 = off0vec[r16]
        off1 = off1vec[r16]
        t65 = tok * (_E + 1)
        for j in range(_E // 16):
          tm_v[pl.ds(t65 + j * 16, 16)] = (
              w0_v[b, tok, pl.ds(off0 + j * 16, 16)] * p0s
              + w1_v[b, tok, pl.ds(off1 + j * 16, 16)] * p1s)
      return carry2

    lax.fori_loop(0, _CHUNK // 16, tok_group, 0)

    def e_group(g, carry2):
      base = g * 16
      toks65 = (iota16 + base) * (_E + 1)
      sbase = boff + base

      @plsc.parallel_loop(0, _E, step=1, unroll=4)
      def _(e):
        ge = plsc.load_gather(tm_v, [toks65 + e])
        outb_v[pl.ds(sbase + e * 128, 16)] = ge

      pb = boff + 8 * 1024 + base
      outb_v[pl.ds(pb, 16)] = pv0_v[b, pl.ds(base, 16)]
      outb_v[pl.ds(pb + 128, 16)] = pv1_v[b, pl.ds(base, 16)]
      return carry2

    lax.fori_loop(0, _CHUNK // 16, e_group, 0)

  start_i(jnp.int32(0))
  start_i(jnp.int32(1))
  start_i(jnp.int32(2))
  wait_i()
  start_a(jnp.int32(0))
  wait_i()
  start_a(jnp.int32(1))
  wait_a()
  start_b(jnp.int32(0))

  def chunk_body(c, carry):
    @pl.when(c < _CPW - 3)
    def _():
      start_i(c + 3)

    @pl.when(c < _CPW - 2)
    def _():
      wait_i()
      start_a(c + 2)

    @pl.when(c < _CPW - 1)
    def _():
      wait_a()
      start_b(c + 1)

    wait_b()

    @pl.when(c >= 2)
    def _():
      wait_d()

    compute(c)
    start_d(c)
    return carry

  lax.fori_loop(0, _CPW, chunk_body, 0)
  wait_d()
  wait_d()


def kernel(words_as_ids, hash_table, W, P):
  ids = words_as_ids.astype(jnp.int32).T.reshape(_NCHUNK, _CHUNK)
  hcol0 = hash_table[:, 0].astype(jnp.int32)
  hcol1 = hash_table[:, 1].astype(jnp.int32)
  pcol0 = P[:, 0]
  pcol1 = P[:, 1]
  w2 = W.reshape(W.shape[0] // 2, 2 * _E)

  mesh = plsc.VectorSubcoreMesh(core_axis_name="c", subcore_axis_name="s")
  run = pl.kernel(
      _hash_embed_body,
      out_type=jax.ShapeDtypeStruct((_L * _LSTRIDE,), jnp.float32),
      mesh=mesh,
      compiler_params=pltpu.CompilerParams(needs_layout_passes=False),
      scratch_types=[
          pltpu.VMEM((3, _CHUNK), jnp.int32),
          pltpu.VMEM((3, _CHUNK), jnp.int32),
          pltpu.VMEM((3, _CHUNK), jnp.int32),
          pltpu.VMEM((3, _CHUNK), jnp.int32),
          pltpu.VMEM((3, _CHUNK), jnp.int32),
          pltpu.VMEM((3, _CHUNK), jnp.float32),
          pltpu.VMEM((3, _CHUNK), jnp.float32),
          pltpu.VMEM((2, _CHUNK), jnp.float32),
          pltpu.VMEM((2, _CHUNK), jnp.float32),
          pltpu.VMEM((2, _CHUNK, 2 * _E), jnp.float32),
          pltpu.VMEM((2, _CHUNK, 2 * _E), jnp.float32),
          pltpu.VMEM((_CHUNK * (_E + 1),), jnp.float32),
          pltpu.VMEM((2 * _OSLAB,), jnp.float32),
          pltpu.SemaphoreType.DMA,
          pltpu.SemaphoreType.DMA,
          pltpu.SemaphoreType.DMA,
          pltpu.SemaphoreType.DMA,
      ],
  )
  raw = run(ids, hcol0, hcol1, w2, pcol0, pcol1)
  out = raw.reshape(_L, _NET, _KTILES, 8, _CHUNK)
  out = out.transpose(2, 4, 0, 1, 3).reshape(_B, _L, _NET * 8)
  return out[:, :, :_E + 2]


# speedup vs baseline: 1.6882x; 1.0311x over previous
import jax
import jax.numpy as jnp
from jax import lax
from jax.experimental import pallas as pl
from jax.experimental.pallas import tpu as pltpu
from jax.experimental.pallas import tpu_sc as plsc

_B = 16384
_L = 20
_E = 64
_N = _B * _L
_CHUNK = 128
_NW = 32
_NCHUNK = _N // _CHUNK
_CPW = _NCHUNK // _NW
_NET = 9
_WPAD = 2 * _E + 4
_OSLAB = 8 * 1024 + 2 * 128
_KTILES = _B // 128
_LSTRIDE = _NET * _KTILES * 1024


def _hash_embed_body(ids_hbm, hcol0_hbm, hcol1_hbm, w_hbm, pcol0_hbm,
                     pcol1_hbm, out_hbm,
                     ids3_v, h0_v, h1_v, p0_v, p1_v,
                     pv0_v, pv1_v, w0_v, w1_v, tm_v, outb_v,
                     sem_i, sem_a, sem_b, sem_out):
  wid = lax.axis_index("s") * 2 + lax.axis_index("c")
  row0 = wid * _CPW

  iota16 = lax.iota(jnp.int32, 16)

  def start_i(c):
    pltpu.async_copy(ids_hbm.at[row0 + c], ids3_v.at[lax.rem(c, 3)], sem_i)

  def wait_i():
    pltpu.make_async_copy(ids_hbm.at[0], ids3_v.at[0], sem_i).wait()

  def start_a(c):
    s = lax.rem(c, 3)
    idx = ids3_v.at[s]
    pltpu.async_copy(hcol0_hbm.at[idx], h0_v.at[s], sem_a)
    pltpu.async_copy(hcol1_hbm.at[idx], h1_v.at[s], sem_a)
    pltpu.async_copy(pcol0_hbm.at[idx], p0_v.at[s], sem_a)
    pltpu.async_copy(pcol1_hbm.at[idx], p1_v.at[s], sem_a)

  def wait_a():
    idx = ids3_v.at[0]
    pltpu.make_async_copy(hcol0_hbm.at[idx], h0_v.at[0], sem_a).wait()
    pltpu.make_async_copy(hcol1_hbm.at[idx], h1_v.at[0], sem_a).wait()
    pltpu.make_async_copy(pcol0_hbm.at[idx], p0_v.at[0], sem_a).wait()
    pltpu.make_async_copy(pcol1_hbm.at[idx], p1_v.at[0], sem_a).wait()

  def start_b(c):
    s = lax.rem(c, 3)
    b = c & 1
    pltpu.async_copy(w_hbm.at[h0_v.at[s]], w0_v.at[b], sem_b)
    pltpu.async_copy(w_hbm.at[h1_v.at[s]], w1_v.at[b], sem_b)
    pltpu.async_copy(pcol0_hbm.at[h0_v.at[s]], pv0_v.at[b], sem_b)
    pltpu.async_copy(pcol1_hbm.at[h1_v.at[s]], pv1_v.at[b], sem_b)

  def wait_b():
    pltpu.make_async_copy(w_hbm.at[h0_v.at[0]], w0_v.at[0], sem_b).wait()
    pltpu.make_async_copy(w_hbm.at[h1_v.at[0]], w1_v.at[0], sem_b).wait()
    pltpu.make_async_copy(pcol0_hbm.at[h0_v.at[0]], pv0_v.at[0], sem_b).wait()
    pltpu.make_async_copy(pcol1_hbm.at[h1_v.at[0]], pv1_v.at[0], sem_b).wait()

  def start_d(c):
    boff = (c & 1) * _OSLAB
    r = row0 + c
    l = lax.div(r, _KTILES)
    k = lax.rem(r, _KTILES)
    base = l * _LSTRIDE + k * 1024
    for t in range(8):
      pltpu.async_copy(outb_v.at[pl.ds(boff + t * 1024, 1024)],
                       out_hbm.at[pl.ds(base + t * 131072, 1024)], sem_out)
    pltpu.async_copy(outb_v.at[pl.ds(boff + 8 * 1024, 256)],
                     out_hbm.at[pl.ds(base + 8 * 131072, 256)], sem_out)

  def wait_d():
    for t in range(8):
      pltpu.make_async_copy(outb_v.at[pl.ds(0, 1024)],
                            out_hbm.at[pl.ds(0, 1024)], sem_out).wait()
    pltpu.make_async_copy(outb_v.at[pl.ds(0, 256)],
                          out_hbm.at[pl.ds(0, 256)], sem_out).wait()

  def compute(c):
    s = lax.rem(c, 3)
    b = c & 1
    boff = b * _OSLAB

    def tok_group(g, carry2):
      base = g * 16
      p0vec = p0_v[s, pl.ds(base, 16)]
      p1vec = p1_v[s, pl.ds(base, 16)]
      for r16 in range(16):
        tok = base + r16
        p0s = p0vec[r16]
        p1s = p1vec[r16]
        t65 = tok * (_E + 1)
        for j in range(_E // 16):
          tm_v[pl.ds(t65 + j * 16, 16)] = (
              w0_v[b, tok, pl.ds(j * 16, 16)] * p0s
              + w1_v[b, tok, pl.ds(j * 16, 16)] * p1s)
      return carry2

    lax.fori_loop(0, _CHUNK // 16, tok_group, 0)

    def e_group(g, carry2):
      base = g * 16
      toks65 = (iota16 + base) * (_E + 1)
      sbase = boff + base

      @plsc.parallel_loop(0, _E, step=1, unroll=4)
      def _(e):
        ge = plsc.load_gather(tm_v, [toks65 + e])
        outb_v[pl.ds(sbase + e * 128, 16)] = ge

      pb = boff + 8 * 1024 + base
      outb_v[pl.ds(pb, 16)] = pv0_v[b, pl.ds(base, 16)]
      outb_v[pl.ds(pb + 128, 16)] = pv1_v[b, pl.ds(base, 16)]
      return carry2

    lax.fori_loop(0, _CHUNK // 16, e_group, 0)

  start_i(jnp.int32(0))
  start_i(jnp.int32(1))
  start_i(jnp.int32(2))
  wait_i()
  start_a(jnp.int32(0))
  wait_i()
  start_a(jnp.int32(1))
  wait_a()
  start_b(jnp.int32(0))

  def chunk_body(c, carry):
    @pl.when(c < _CPW - 3)
    def _():
      start_i(c + 3)

    @pl.when(c < _CPW - 2)
    def _():
      wait_i()
      start_a(c + 2)

    @pl.when(c < _CPW - 1)
    def _():
      wait_a()
      start_b(c + 1)

    wait_b()

    @pl.when(c >= 2)
    def _():
      wait_d()

    compute(c)
    start_d(c)
    return carry

  lax.fori_loop(0, _CPW, chunk_body, 0)
  wait_d()
  wait_d()


def kernel(words_as_ids, hash_table, W, P):
  ids = words_as_ids.astype(jnp.int32).T.reshape(_NCHUNK, _CHUNK)
  hcol0 = hash_table[:, 0].astype(jnp.int32)
  hcol1 = hash_table[:, 1].astype(jnp.int32)
  pcol0 = P[:, 0]
  pcol1 = P[:, 1]
  w2 = W

  mesh = plsc.VectorSubcoreMesh(core_axis_name="c", subcore_axis_name="s")
  run = pl.kernel(
      _hash_embed_body,
      out_type=jax.ShapeDtypeStruct((_L * _LSTRIDE,), jnp.float32),
      mesh=mesh,
      compiler_params=pltpu.CompilerParams(needs_layout_passes=False,
                                           use_tc_tiling_on_sc=False),
      scratch_types=[
          pltpu.VMEM((3, _CHUNK), jnp.int32),
          pltpu.VMEM((3, _CHUNK), jnp.int32),
          pltpu.VMEM((3, _CHUNK), jnp.int32),
          pltpu.VMEM((3, _CHUNK), jnp.float32),
          pltpu.VMEM((3, _CHUNK), jnp.float32),
          pltpu.VMEM((2, _CHUNK), jnp.float32),
          pltpu.VMEM((2, _CHUNK), jnp.float32),
          pltpu.VMEM((2, _CHUNK, _E), jnp.float32),
          pltpu.VMEM((2, _CHUNK, _E), jnp.float32),
          pltpu.VMEM((_CHUNK * (_E + 1),), jnp.float32),
          pltpu.VMEM((2 * _OSLAB,), jnp.float32),
          pltpu.SemaphoreType.DMA,
          pltpu.SemaphoreType.DMA,
          pltpu.SemaphoreType.DMA,
          pltpu.SemaphoreType.DMA,
      ],
  )
  raw = run(ids, hcol0, hcol1, w2, pcol0, pcol1)
  out = raw.reshape(_L, _NET, _KTILES, 8, _CHUNK)
  out = out.transpose(2, 4, 0, 1, 3).reshape(_B, _L, _NET * 8)
  return out[:, :, :_E + 2]


# speedup vs baseline: 2.2963x; 1.3602x over previous
import jax
import jax.numpy as jnp
from jax import lax
from jax.experimental import pallas as pl
from jax.experimental.pallas import tpu as pltpu
from jax.experimental.pallas import tpu_sc as plsc

_B = 16384
_L = 20
_E = 64
_N = _B * _L
_CHUNK = 128
_NW = 32
_NCHUNK = _N // _CHUNK
_CPW = _NCHUNK // _NW
_NET = 9
_WPAD = 2 * _E + 4
_OSLAB = 8 * 1024 + 2 * 128
_KTILES = _B // 128
_LSTRIDE = _NET * _KTILES * 1024


def _hash_embed_body(ids_hbm, hcol0_hbm, hcol1_hbm, w_hbm, pcol0_hbm,
                     pcol1_hbm, out_hbm,
                     ids3_v, h0_v, h1_v, p0_v, p1_v,
                     pv0_v, pv1_v, w0_v, w1_v, tm_v, outb_v,
                     sem_i, sem_a, sem_b, sem_out):
  wid = lax.axis_index("s") * 2 + lax.axis_index("c")
  row0 = wid * _CPW

  iota16 = lax.iota(jnp.int32, 16)

  def start_i(c):
    pltpu.async_copy(ids_hbm.at[row0 + c], ids3_v.at[lax.rem(c, 3)], sem_i)

  def wait_i():
    pltpu.make_async_copy(ids_hbm.at[0], ids3_v.at[0], sem_i).wait()

  def start_a(c):
    s = lax.rem(c, 3)
    idx = ids3_v.at[s]
    pltpu.async_copy(hcol0_hbm.at[idx], h0_v.at[s], sem_a)
    pltpu.async_copy(hcol1_hbm.at[idx], h1_v.at[s], sem_a)
    pltpu.async_copy(pcol0_hbm.at[idx], p0_v.at[s], sem_a)
    pltpu.async_copy(pcol1_hbm.at[idx], p1_v.at[s], sem_a)

  def wait_a():
    idx = ids3_v.at[0]
    pltpu.make_async_copy(hcol0_hbm.at[idx], h0_v.at[0], sem_a).wait()
    pltpu.make_async_copy(hcol1_hbm.at[idx], h1_v.at[0], sem_a).wait()
    pltpu.make_async_copy(pcol0_hbm.at[idx], p0_v.at[0], sem_a).wait()
    pltpu.make_async_copy(pcol1_hbm.at[idx], p1_v.at[0], sem_a).wait()

  def start_b(c):
    s = lax.rem(c, 3)
    b = c & 1
    pltpu.async_copy(w_hbm.at[h0_v.at[s]], w0_v.at[b], sem_b)
    pltpu.async_copy(w_hbm.at[h1_v.at[s]], w1_v.at[b], sem_b)
    pltpu.async_copy(pcol0_hbm.at[h0_v.at[s]], pv0_v.at[b], sem_b)
    pltpu.async_copy(pcol1_hbm.at[h1_v.at[s]], pv1_v.at[b], sem_b)

  def wait_b():
    pltpu.make_async_copy(w_hbm.at[h0_v.at[0]], w0_v.at[0], sem_b).wait()
    pltpu.make_async_copy(w_hbm.at[h1_v.at[0]], w1_v.at[0], sem_b).wait()
    pltpu.make_async_copy(pcol0_hbm.at[h0_v.at[0]], pv0_v.at[0], sem_b).wait()
    pltpu.make_async_copy(pcol1_hbm.at[h1_v.at[0]], pv1_v.at[0], sem_b).wait()

  def start_d(c):
    boff = (c & 1) * _OSLAB
    r = row0 + c
    l = lax.div(r, _KTILES)
    k = lax.rem(r, _KTILES)
    base = l * _LSTRIDE + k * 1024
    for t in range(8):
      pltpu.async_copy(outb_v.at[pl.ds(boff + t * 1024, 1024)],
                       out_hbm.at[pl.ds(base + t * 131072, 1024)], sem_out)
    pltpu.async_copy(outb_v.at[pl.ds(boff + 8 * 1024, 256)],
                     out_hbm.at[pl.ds(base + 8 * 131072, 256)], sem_out)

  def wait_d():
    for t in range(8):
      pltpu.make_async_copy(outb_v.at[pl.ds(0, 1024)],
                            out_hbm.at[pl.ds(0, 1024)], sem_out).wait()
    pltpu.make_async_copy(outb_v.at[pl.ds(0, 256)],
                          out_hbm.at[pl.ds(0, 256)], sem_out).wait()

  def compute(c):
    s = lax.rem(c, 3)
    b = c & 1
    boff = b * _OSLAB

    @plsc.parallel_loop(0, _CHUNK // 16, step=1)
    def tok_group(g):
      base = g * 16
      p0vec = p0_v[s, pl.ds(base, 16)]
      p1vec = p1_v[s, pl.ds(base, 16)]
      for r16 in range(16):
        tok = base + r16
        p0s = p0vec[r16]
        p1s = p1vec[r16]
        t65 = tok * (_E + 1)
        for j in range(_E // 16):
          tm_v[pl.ds(t65 + j * 16, 16)] = (
              w0_v[b, tok, pl.ds(j * 16, 16)] * p0s
              + w1_v[b, tok, pl.ds(j * 16, 16)] * p1s)

    def e_group(g, carry2):
      base = g * 16
      toks65 = (iota16 + base) * (_E + 1)
      sbase = boff + base

      @plsc.parallel_loop(0, _E, step=1, unroll=4)
      def _(e):
        ge = plsc.load_gather(tm_v, [toks65 + e])
        outb_v[pl.ds(sbase + e * 128, 16)] = ge

      pb = boff + 8 * 1024 + base
      outb_v[pl.ds(pb, 16)] = pv0_v[b, pl.ds(base, 16)]
      outb_v[pl.ds(pb + 128, 16)] = pv1_v[b, pl.ds(base, 16)]
      return carry2

    lax.fori_loop(0, _CHUNK // 16, e_group, 0)

  start_i(jnp.int32(0))
  start_i(jnp.int32(1))
  start_i(jnp.int32(2))
  wait_i()
  start_a(jnp.int32(0))
  wait_i()
  start_a(jnp.int32(1))
  wait_a()
  start_b(jnp.int32(0))

  def chunk_body(c, carry):
    @pl.when(c < _CPW - 3)
    def _():
      start_i(c + 3)

    @pl.when(c < _CPW - 2)
    def _():
      wait_i()
      start_a(c + 2)

    @pl.when(c < _CPW - 1)
    def _():
      wait_a()
      start_b(c + 1)

    wait_b()

    @pl.when(c >= 2)
    def _():
      wait_d()

    compute(c)
    start_d(c)
    return carry

  lax.fori_loop(0, _CPW, chunk_body, 0)
  wait_d()
  wait_d()


def kernel(words_as_ids, hash_table, W, P):
  ids = words_as_ids.astype(jnp.int32).T.reshape(_NCHUNK, _CHUNK)
  hcol0 = hash_table[:, 0].astype(jnp.int32)
  hcol1 = hash_table[:, 1].astype(jnp.int32)
  pcol0 = P[:, 0]
  pcol1 = P[:, 1]
  w2 = W

  mesh = plsc.VectorSubcoreMesh(core_axis_name="c", subcore_axis_name="s")
  run = pl.kernel(
      _hash_embed_body,
      out_type=jax.ShapeDtypeStruct((_L * _LSTRIDE,), jnp.float32),
      mesh=mesh,
      compiler_params=pltpu.CompilerParams(needs_layout_passes=False,
                                           use_tc_tiling_on_sc=False),
      scratch_types=[
          pltpu.VMEM((3, _CHUNK), jnp.int32),
          pltpu.VMEM((3, _CHUNK), jnp.int32),
          pltpu.VMEM((3, _CHUNK), jnp.int32),
          pltpu.VMEM((3, _CHUNK), jnp.float32),
          pltpu.VMEM((3, _CHUNK), jnp.float32),
          pltpu.VMEM((2, _CHUNK), jnp.float32),
          pltpu.VMEM((2, _CHUNK), jnp.float32),
          pltpu.VMEM((2, _CHUNK, _E), jnp.float32),
          pltpu.VMEM((2, _CHUNK, _E), jnp.float32),
          pltpu.VMEM((_CHUNK * (_E + 1),), jnp.float32),
          pltpu.VMEM((2 * _OSLAB,), jnp.float32),
          pltpu.SemaphoreType.DMA,
          pltpu.SemaphoreType.DMA,
          pltpu.SemaphoreType.DMA,
          pltpu.SemaphoreType.DMA,
      ],
  )
  raw = run(ids, hcol0, hcol1, w2, pcol0, pcol1)
  out = raw.reshape(_L, _NET, _KTILES, 8, _CHUNK)
  out = out.transpose(2, 4, 0, 1, 3).reshape(_B, _L, _NET * 8)
  return out[:, :, :_E + 2]
